# baseline (speedup 1.0000x reference)
import functools

import jax
import jax.numpy as jnp
from jax import lax
from jax.experimental import pallas as pl
from jax.experimental.pallas import tpu as pltpu
from jax.experimental.pallas import tpu_sc as plsc

_TOTAL = 32768
_NB = 16
_L = 16

_info = plsc.get_sparse_core_info()
_NC = _info.num_cores
_NS = _info.num_subcores
_NW = _NC * _NS
_PPW = _TOTAL // _NW

_mesh = plsc.VectorSubcoreMesh(core_axis_name="c", subcore_axis_name="s")


@functools.partial(
    pl.kernel,
    mesh=_mesh,
    out_type=(
        jax.ShapeDtypeStruct((3, _TOTAL), jnp.float32),
        jax.ShapeDtypeStruct((_TOTAL,), jnp.int32),
    ),
    compiler_params=pltpu.CompilerParams(
        needs_layout_passes=False, use_tc_tiling_on_sc=False
    ),
    scratch_types=[
        pltpu.VMEM((_NB * 16,), jnp.float32),
        pltpu.VMEM((3, _PPW), jnp.float32),
        pltpu.VMEM((_PPW,), jnp.int32),
        pltpu.VMEM((3, _PPW), jnp.float32),
    ],
)
def _se3_sc(tr_hbm, pos_hbm, bat_hbm, out_hbm, bat_out_hbm, tr_v, pos_v, bat_v, out_v):
    wid = lax.axis_index("s") * _NC + lax.axis_index("c")
    pbase = wid * _PPW
    pltpu.sync_copy(tr_hbm, tr_v)
    pltpu.sync_copy(pos_hbm.at[:, pl.ds(pbase, _PPW)], pos_v)
    pltpu.sync_copy(bat_hbm.at[pl.ds(pbase, _PPW)], bat_v)

    def body(k, carry):
        p = k * _L
        b = bat_v[pl.ds(p, _L)]
        t = b * 16
        r00 = plsc.load_gather(tr_v, [t])
        r01 = plsc.load_gather(tr_v, [t + 1])
        r02 = plsc.load_gather(tr_v, [t + 2])
        p0 = plsc.load_gather(tr_v, [t + 3])
        r10 = plsc.load_gather(tr_v, [t + 4])
        r11 = plsc.load_gather(tr_v, [t + 5])
        r12 = plsc.load_gather(tr_v, [t + 6])
        p1 = plsc.load_gather(tr_v, [t + 7])
        r20 = plsc.load_gather(tr_v, [t + 8])
        r21 = plsc.load_gather(tr_v, [t + 9])
        r22 = plsc.load_gather(tr_v, [t + 10])
        p2 = plsc.load_gather(tr_v, [t + 11])
        x = pos_v[0, pl.ds(p, _L)]
        y = pos_v[1, pl.ds(p, _L)]
        z = pos_v[2, pl.ds(p, _L)]
        out_v[0, pl.ds(p, _L)] = r00 * x + r01 * y + r02 * z + p0
        out_v[1, pl.ds(p, _L)] = r10 * x + r11 * y + r12 * z + p1
        out_v[2, pl.ds(p, _L)] = r20 * x + r21 * y + r22 * z + p2
        return carry

    lax.fori_loop(0, _PPW // _L, body, 0)
    pltpu.sync_copy(out_v, out_hbm.at[:, pl.ds(pbase, _PPW)])
    pltpu.sync_copy(bat_v, bat_out_hbm.at[pl.ds(pbase, _PPW)])


def kernel(trans, pos, batch):
    outT, new_batch = _se3_sc(trans.reshape(-1), pos.T, batch.astype(jnp.int32))
    return outT.T, new_batch


# speedup vs baseline: 1.0386x; 1.0386x over previous
import functools

import jax
import jax.numpy as jnp
from jax import lax
from jax.experimental import pallas as pl
from jax.experimental.pallas import tpu as pltpu
from jax.experimental.pallas import tpu_sc as plsc

_TOTAL = 32768
_NB = 16
_L = 16

_info = plsc.get_sparse_core_info()
_NC = _info.num_cores
_NS = _info.num_subcores
_NW = 1 * _NS
_PPW = _TOTAL // _NW

_mesh = plsc.VectorSubcoreMesh(core_axis_name="c", subcore_axis_name="s", num_cores=1)


@functools.partial(
    pl.kernel,
    mesh=_mesh,
    out_type=(
        jax.ShapeDtypeStruct((3, _TOTAL), jnp.float32),
        jax.ShapeDtypeStruct((_TOTAL,), jnp.int32),
    ),
    compiler_params=pltpu.CompilerParams(
        needs_layout_passes=False, use_tc_tiling_on_sc=False
    ),
    scratch_types=[
        pltpu.VMEM((_NB * 16,), jnp.float32),
        pltpu.VMEM((3, _PPW), jnp.float32),
        pltpu.VMEM((_PPW,), jnp.int32),
        pltpu.VMEM((3, _PPW), jnp.float32),
    ],
)
def _se3_sc(tr_hbm, pos_hbm, bat_hbm, out_hbm, bat_out_hbm, tr_v, pos_v, bat_v, out_v):
    wid = lax.axis_index("s")
    pbase = wid * _PPW
    pltpu.sync_copy(tr_hbm, tr_v)
    pltpu.sync_copy(pos_hbm.at[:, pl.ds(pbase, _PPW)], pos_v)
    pltpu.sync_copy(bat_hbm.at[pl.ds(pbase, _PPW)], bat_v)

    def body(k, carry):
        p = k * _L
        b = bat_v[pl.ds(p, _L)]
        t = b * 16
        r00 = plsc.load_gather(tr_v, [t])
        r01 = plsc.load_gather(tr_v, [t + 1])
        r02 = plsc.load_gather(tr_v, [t + 2])
        p0 = plsc.load_gather(tr_v, [t + 3])
        r10 = plsc.load_gather(tr_v, [t + 4])
        r11 = plsc.load_gather(tr_v, [t + 5])
        r12 = plsc.load_gather(tr_v, [t + 6])
        p1 = plsc.load_gather(tr_v, [t + 7])
        r20 = plsc.load_gather(tr_v, [t + 8])
        r21 = plsc.load_gather(tr_v, [t + 9])
        r22 = plsc.load_gather(tr_v, [t + 10])
        p2 = plsc.load_gather(tr_v, [t + 11])
        x = pos_v[0, pl.ds(p, _L)]
        y = pos_v[1, pl.ds(p, _L)]
        z = pos_v[2, pl.ds(p, _L)]
        out_v[0, pl.ds(p, _L)] = r00 * x + r01 * y + r02 * z + p0
        out_v[1, pl.ds(p, _L)] = r10 * x + r11 * y + r12 * z + p1
        out_v[2, pl.ds(p, _L)] = r20 * x + r21 * y + r22 * z + p2
        return carry

    lax.fori_loop(0, _PPW // _L, body, 0)
    pltpu.sync_copy(out_v, out_hbm.at[:, pl.ds(pbase, _PPW)])
    pltpu.sync_copy(bat_v, bat_out_hbm.at[pl.ds(pbase, _PPW)])


def kernel(trans, pos, batch):
    outT, new_batch = _se3_sc(trans.reshape(-1), pos.T, batch.astype(jnp.int32))
    return outT.T, new_batch


# speedup vs baseline: 1.1118x; 1.0705x over previous
import functools

import jax
import jax.numpy as jnp
from jax import lax
from jax.experimental import pallas as pl
from jax.experimental.pallas import tpu as pltpu
from jax.experimental.pallas import tpu_sc as plsc

_TOTAL = 32768
_NB = 16
_L = 16

_info = plsc.get_sparse_core_info()
_NS = _info.num_subcores
_NW = _NS
_PPW = _TOTAL // _NW

_mesh = plsc.VectorSubcoreMesh(
    core_axis_name="c", subcore_axis_name="s", num_cores=1
)


@functools.partial(
    pl.kernel,
    mesh=_mesh,
    out_type=(
        jax.ShapeDtypeStruct((3, _TOTAL), jnp.float32),
        jax.ShapeDtypeStruct((_TOTAL,), jnp.int32),
    ),
    compiler_params=pltpu.CompilerParams(
        needs_layout_passes=False, use_tc_tiling_on_sc=False
    ),
    scratch_types=[
        pltpu.VMEM((_NB * 16,), jnp.float32),
        pltpu.VMEM((3, _PPW), jnp.float32),
        pltpu.VMEM((_PPW,), jnp.int32),
        pltpu.VMEM((3, _PPW), jnp.float32),
        [pltpu.VMEM((_NB,), jnp.float32) for _ in range(12)],
        pltpu.SemaphoreType.DMA,
        pltpu.SemaphoreType.DMA,
        pltpu.SemaphoreType.DMA,
    ],
)
def _se3_sc(
    tr_hbm, pos_hbm, bat_hbm, out_hbm, bat_out_hbm,
    tr_v, pos_v, bat_v, out_v, tabs, sem_t, sem_p, sem_b,
):
    wid = lax.axis_index("s")
    pbase = wid * _PPW
    cp_t = pltpu.async_copy(tr_hbm, tr_v, sem_t)
    cp_p = pltpu.async_copy(pos_hbm.at[:, pl.ds(pbase, _PPW)], pos_v, sem_p)
    cp_b = pltpu.async_copy(bat_hbm.at[pl.ds(pbase, _PPW)], bat_v, sem_b)
    cp_t.wait()

    iota = lax.iota(jnp.int32, _L)
    for c in range(12):
        tabs[c][...] = plsc.load_gather(tr_v, [iota * 16 + c])
    cp_b.wait()
    cp_p.wait()

    @plsc.parallel_loop(0, _PPW // _L, unroll=2)
    def body(k):
        p = k * _L
        b = bat_v[pl.ds(p, _L)]
        r00 = plsc.load_gather(tabs[0], [b])
        r01 = plsc.load_gather(tabs[1], [b])
        r02 = plsc.load_gather(tabs[2], [b])
        p0 = plsc.load_gather(tabs[3], [b])
        r10 = plsc.load_gather(tabs[4], [b])
        r11 = plsc.load_gather(tabs[5], [b])
        r12 = plsc.load_gather(tabs[6], [b])
        p1 = plsc.load_gather(tabs[7], [b])
        r20 = plsc.load_gather(tabs[8], [b])
        r21 = plsc.load_gather(tabs[9], [b])
        r22 = plsc.load_gather(tabs[10], [b])
        p2 = plsc.load_gather(tabs[11], [b])
        x = pos_v[0, pl.ds(p, _L)]
        y = pos_v[1, pl.ds(p, _L)]
        z = pos_v[2, pl.ds(p, _L)]
        out_v[0, pl.ds(p, _L)] = r00 * x + r01 * y + r02 * z + p0
        out_v[1, pl.ds(p, _L)] = r10 * x + r11 * y + r12 * z + p1
        out_v[2, pl.ds(p, _L)] = r20 * x + r21 * y + r22 * z + p2

    cp_o = pltpu.async_copy(out_v, out_hbm.at[:, pl.ds(pbase, _PPW)], sem_p)
    cp_bo = pltpu.async_copy(bat_v, bat_out_hbm.at[pl.ds(pbase, _PPW)], sem_b)
    cp_o.wait()
    cp_bo.wait()


def kernel(trans, pos, batch):
    outT, new_batch = _se3_sc(trans.reshape(-1), pos.T, batch.astype(jnp.int32))
    return outT.T, new_batch


# speedup vs baseline: 1.1179x; 1.0054x over previous
import functools

import jax
import jax.numpy as jnp
from jax import lax
from jax.experimental import pallas as pl
from jax.experimental.pallas import tpu as pltpu
from jax.experimental.pallas import tpu_sc as plsc

_TOTAL = 32768
_NB = 16
_L = 16

_info = plsc.get_sparse_core_info()
_NS = _info.num_subcores
_NW = _NS
_PPW = _TOTAL // _NW

_mesh = plsc.VectorSubcoreMesh(
    core_axis_name="c", subcore_axis_name="s", num_cores=1
)


@functools.partial(
    pl.kernel,
    mesh=_mesh,
    out_type=(
        jax.ShapeDtypeStruct((3, _TOTAL), jnp.float32),
        jax.ShapeDtypeStruct((_TOTAL,), jnp.int32),
    ),
    compiler_params=pltpu.CompilerParams(
        needs_layout_passes=False, use_tc_tiling_on_sc=False
    ),
    scratch_types=[
        pltpu.VMEM((_NB * 16,), jnp.float32),
        pltpu.VMEM((3, _PPW), jnp.float32),
        pltpu.VMEM((_PPW,), jnp.int32),
        pltpu.VMEM((3, _PPW), jnp.float32),
        [pltpu.VMEM((_NB,), jnp.float32) for _ in range(12)],
        pltpu.SemaphoreType.DMA,
        pltpu.SemaphoreType.DMA,
        pltpu.SemaphoreType.DMA,
    ],
)
def _se3_sc(
    tr_hbm, pos_hbm, bat_hbm, out_hbm, bat_out_hbm,
    tr_v, pos_v, bat_v, out_v, tabs, sem_t, sem_p, sem_b,
):
    wid = lax.axis_index("s")
    pbase = wid * _PPW
    cp_t = pltpu.async_copy(tr_hbm, tr_v, sem_t)
    cp_p = pltpu.async_copy(pos_hbm.at[:, pl.ds(pbase, _PPW)], pos_v, sem_p)
    cp_b = pltpu.async_copy(bat_hbm.at[pl.ds(pbase, _PPW)], bat_v, sem_b)
    cp_t.wait()

    iota = lax.iota(jnp.int32, _L)
    for c in range(12):
        tabs[c][...] = plsc.load_gather(tr_v, [iota + c * 16])
    cp_b.wait()
    cp_bo = pltpu.async_copy(bat_v, bat_out_hbm.at[pl.ds(pbase, _PPW)], sem_b)
    cp_p.wait()

    @plsc.parallel_loop(0, _PPW // _L, unroll=2)
    def body(k):
        p = k * _L
        b = bat_v[pl.ds(p, _L)]
        r00 = plsc.load_gather(tabs[0], [b])
        r01 = plsc.load_gather(tabs[1], [b])
        r02 = plsc.load_gather(tabs[2], [b])
        p0 = plsc.load_gather(tabs[3], [b])
        r10 = plsc.load_gather(tabs[4], [b])
        r11 = plsc.load_gather(tabs[5], [b])
        r12 = plsc.load_gather(tabs[6], [b])
        p1 = plsc.load_gather(tabs[7], [b])
        r20 = plsc.load_gather(tabs[8], [b])
        r21 = plsc.load_gather(tabs[9], [b])
        r22 = plsc.load_gather(tabs[10], [b])
        p2 = plsc.load_gather(tabs[11], [b])
        x = pos_v[0, pl.ds(p, _L)]
        y = pos_v[1, pl.ds(p, _L)]
        z = pos_v[2, pl.ds(p, _L)]
        out_v[0, pl.ds(p, _L)] = r00 * x + r01 * y + r02 * z + p0
        out_v[1, pl.ds(p, _L)] = r10 * x + r11 * y + r12 * z + p1
        out_v[2, pl.ds(p, _L)] = r20 * x + r21 * y + r22 * z + p2

    cp_o = pltpu.async_copy(out_v, out_hbm.at[:, pl.ds(pbase, _PPW)], sem_p)
    cp_o.wait()
    cp_bo.wait()


def kernel(trans, pos, batch):
    trc = trans.transpose(1, 2, 0).reshape(-1)
    outT, new_batch = _se3_sc(trc, pos.T, batch.astype(jnp.int32))
    return outT.T, new_batch


# speedup vs baseline: 1.1193x; 1.0013x over previous
import functools

import jax
import jax.numpy as jnp
from jax import lax
from jax.experimental import pallas as pl
from jax.experimental.pallas import tpu as pltpu
from jax.experimental.pallas import tpu_sc as plsc

_TOTAL = 32768
_NB = 16
_L = 16

_info = plsc.get_sparse_core_info()
_NS = _info.num_subcores
_NW = _NS
_PPW = _TOTAL // _NW

_mesh = plsc.VectorSubcoreMesh(
    core_axis_name="c", subcore_axis_name="s", num_cores=1
)


@functools.partial(
    pl.kernel,
    mesh=_mesh,
    out_type=(
        jax.ShapeDtypeStruct((_TOTAL // 128, 4, 128), jnp.float32),
        jax.ShapeDtypeStruct((_TOTAL,), jnp.int32),
    ),
    compiler_params=pltpu.CompilerParams(
        needs_layout_passes=False, use_tc_tiling_on_sc=False
    ),
    scratch_types=[
        pltpu.VMEM((_NB * 16,), jnp.float32),
        pltpu.VMEM((3, _PPW), jnp.float32),
        pltpu.VMEM((_PPW,), jnp.int32),
        pltpu.VMEM((_PPW // 128, 4, 128), jnp.float32),
        [pltpu.VMEM((_NB,), jnp.float32) for _ in range(12)],
        pltpu.SemaphoreType.DMA,
        pltpu.SemaphoreType.DMA,
        pltpu.SemaphoreType.DMA,
    ],
)
def _se3_sc(
    tr_hbm, pos_hbm, bat_hbm, out_hbm, bat_out_hbm,
    tr_v, pos_v, bat_v, out_v, tabs, sem_t, sem_p, sem_b,
):
    wid = lax.axis_index("s")
    pbase = wid * _PPW
    cp_t = pltpu.async_copy(tr_hbm, tr_v, sem_t)
    cp_p = pltpu.async_copy(pos_hbm.at[:, pl.ds(pbase, _PPW)], pos_v, sem_p)
    cp_b = pltpu.async_copy(bat_hbm.at[pl.ds(pbase, _PPW)], bat_v, sem_b)
    cp_t.wait()

    iota = lax.iota(jnp.int32, _L)
    for c in range(12):
        tabs[c][...] = plsc.load_gather(tr_v, [iota + c * 16])
    cp_b.wait()
    cp_bo = pltpu.async_copy(bat_v, bat_out_hbm.at[pl.ds(pbase, _PPW)], sem_b)
    cp_p.wait()

    @plsc.parallel_loop(0, _PPW // _L, unroll=2)
    def body(k):
        p = k * _L
        b = bat_v[pl.ds(p, _L)]
        r00 = plsc.load_gather(tabs[0], [b])
        r01 = plsc.load_gather(tabs[1], [b])
        r02 = plsc.load_gather(tabs[2], [b])
        p0 = plsc.load_gather(tabs[3], [b])
        r10 = plsc.load_gather(tabs[4], [b])
        r11 = plsc.load_gather(tabs[5], [b])
        r12 = plsc.load_gather(tabs[6], [b])
        p1 = plsc.load_gather(tabs[7], [b])
        r20 = plsc.load_gather(tabs[8], [b])
        r21 = plsc.load_gather(tabs[9], [b])
        r22 = plsc.load_gather(tabs[10], [b])
        p2 = plsc.load_gather(tabs[11], [b])
        x = pos_v[0, pl.ds(p, _L)]
        y = pos_v[1, pl.ds(p, _L)]
        z = pos_v[2, pl.ds(p, _L)]
        j = k >> 3
        o = (k & 7) * _L
        out_v[j, 0, pl.ds(o, _L)] = r00 * x + r01 * y + r02 * z + p0
        out_v[j, 1, pl.ds(o, _L)] = r10 * x + r11 * y + r12 * z + p1
        out_v[j, 2, pl.ds(o, _L)] = r20 * x + r21 * y + r22 * z + p2

    cp_o = pltpu.async_copy(out_v, out_hbm.at[pl.ds(wid * (_PPW // 128), _PPW // 128)], sem_p)
    cp_o.wait()
    cp_bo.wait()


def kernel(trans, pos, batch):
    trc = trans.transpose(1, 2, 0).reshape(-1)
    out3d, new_batch = _se3_sc(trc, pos.T, batch.astype(jnp.int32))
    out = out3d.transpose(1, 0, 2).reshape(4, _TOTAL)[:3].T
    return out, new_batch


# speedup vs baseline: 1.1197x; 1.0004x over previous
import functools

import jax
import jax.numpy as jnp
from jax import lax
from jax.experimental import pallas as pl
from jax.experimental.pallas import tpu as pltpu
from jax.experimental.pallas import tpu_sc as plsc

_TOTAL = 32768
_NB = 16
_L = 16

_info = plsc.get_sparse_core_info()
_NS = _info.num_subcores
_NW = _NS
_PPW = _TOTAL // _NW

_mesh = plsc.VectorSubcoreMesh(
    core_axis_name="c", subcore_axis_name="s", num_cores=1
)


@functools.partial(
    pl.kernel,
    mesh=_mesh,
    out_type=(
        jax.ShapeDtypeStruct((3, _TOTAL), jnp.float32),
        jax.ShapeDtypeStruct((_TOTAL,), jnp.int32),
    ),
    compiler_params=pltpu.CompilerParams(
        needs_layout_passes=False, use_tc_tiling_on_sc=False
    ),
    scratch_types=[
        pltpu.VMEM((_NB * 16,), jnp.float32),
        pltpu.VMEM((3, _PPW), jnp.float32),
        pltpu.VMEM((_PPW,), jnp.int32),
        pltpu.VMEM((3, _PPW), jnp.float32),
        [pltpu.VMEM((_NB,), jnp.float32) for _ in range(12)],
        pltpu.SemaphoreType.DMA,
        pltpu.SemaphoreType.DMA,
        pltpu.SemaphoreType.DMA,
    ],
)
def _se3_sc(
    tr_hbm, pos_hbm, bat_hbm, out_hbm, bat_out_hbm,
    tr_v, pos_v, bat_v, out_v, tabs, sem_t, sem_p, sem_b,
):
    wid = lax.axis_index("s")
    pbase = wid * _PPW
    cp_t = pltpu.async_copy(tr_hbm, tr_v, sem_t)
    cp_p = pltpu.async_copy(pos_hbm.at[:, pl.ds(pbase, _PPW)], pos_v, sem_p)
    cp_b = pltpu.async_copy(bat_hbm.at[pl.ds(pbase, _PPW)], bat_v, sem_b)
    cp_t.wait()

    iota = lax.iota(jnp.int32, _L)
    for c in range(12):
        tabs[c][...] = plsc.load_gather(tr_v, [iota + c * 16])
    cp_b.wait()
    cp_bo = pltpu.async_copy(bat_v, bat_out_hbm.at[pl.ds(pbase, _PPW)], sem_b)
    cp_p.wait()

    @plsc.parallel_loop(0, _PPW // _L, unroll=2)
    def body(k):
        p = k * _L
        b = bat_v[pl.ds(p, _L)]
        r00 = plsc.load_gather(tabs[0], [b])
        r01 = plsc.load_gather(tabs[1], [b])
        r02 = plsc.load_gather(tabs[2], [b])
        p0 = plsc.load_gather(tabs[3], [b])
        r10 = plsc.load_gather(tabs[4], [b])
        r11 = plsc.load_gather(tabs[5], [b])
        r12 = plsc.load_gather(tabs[6], [b])
        p1 = plsc.load_gather(tabs[7], [b])
        r20 = plsc.load_gather(tabs[8], [b])
        r21 = plsc.load_gather(tabs[9], [b])
        r22 = plsc.load_gather(tabs[10], [b])
        p2 = plsc.load_gather(tabs[11], [b])
        x = pos_v[0, pl.ds(p, _L)]
        y = pos_v[1, pl.ds(p, _L)]
        z = pos_v[2, pl.ds(p, _L)]
        out_v[0, pl.ds(p, _L)] = r00 * x + r01 * y + r02 * z + p0
        out_v[1, pl.ds(p, _L)] = r10 * x + r11 * y + r12 * z + p1
        out_v[2, pl.ds(p, _L)] = r20 * x + r21 * y + r22 * z + p2

    cp_o = pltpu.async_copy(out_v, out_hbm.at[:, pl.ds(pbase, _PPW)], sem_p)
    cp_o.wait()
    cp_bo.wait()


def kernel(trans, pos, batch):
    trc = trans.transpose(1, 2, 0).reshape(-1)
    outT, new_batch = _se3_sc(trc, pos.T, batch.astype(jnp.int32))
    return outT.T, new_batch
